# baseline (speedup 1.0000x reference)
import functools

import jax
import jax.numpy as jnp
from jax import lax
from jax.experimental import pallas as pl
from jax.experimental.pallas import tpu as pltpu
from jax.experimental.pallas import tpu_sc as plsc

NC, NS = 2, 16
NW = NC * NS
L2W = 1e-06


def _sc_mesh():
    return plsc.VectorSubcoreMesh(core_axis_name="c", subcore_axis_name="s",
                                  num_cores=NC, num_subcores=NS)


_SC_PARAMS = pltpu.CompilerParams(use_tc_tiling_on_sc=False)
_SC_PARAMS_V = pltpu.CompilerParams(use_tc_tiling_on_sc=False,
                                    needs_layout_passes=False)


def _wid():
    return lax.axis_index("s") * NC + lax.axis_index("c")


def _sc_stage1(head, adj_e, adj_r):
    Bn = head.shape[0]
    nn = adj_e.shape[1]
    bpw = Bn // NW

    out_type = (
        jax.ShapeDtypeStruct((Bn, nn), jnp.int32),
        jax.ShapeDtypeStruct((Bn, nn), jnp.int32),
    )

    @functools.partial(
        pl.kernel, out_type=out_type, mesh=_sc_mesh(),
        compiler_params=_SC_PARAMS,
        scratch_types=[
            pltpu.VMEM((bpw,), jnp.int32),
            pltpu.VMEM((bpw, nn), jnp.int32),
            pltpu.SemaphoreType.DMA,
        ],
    )
    def k(head_h, adj_e_h, adj_r_h, e1_h, r1_h, idx_v, rows_i, sem):
        base = _wid() * bpw
        sl = pl.ds(base, bpw)
        pltpu.sync_copy(head_h.at[sl], idx_v)
        pltpu.async_copy(adj_e_h.at[idx_v], rows_i, sem).wait()
        pltpu.sync_copy(rows_i, e1_h.at[sl])
        pltpu.async_copy(adj_r_h.at[idx_v], rows_i, sem).wait()
        pltpu.sync_copy(rows_i, r1_h.at[sl])

    return k(head, adj_e, adj_r)


def _sc_stage_u(uidx, usr_emb):
    Bn = uidx.shape[0]
    dim = usr_emb.shape[1]
    bpw = Bn // NW

    out_type = jax.ShapeDtypeStruct((Bn, dim), jnp.float32)

    @functools.partial(
        pl.kernel, out_type=out_type, mesh=_sc_mesh(),
        compiler_params=_SC_PARAMS,
        scratch_types=[
            pltpu.VMEM((bpw,), jnp.int32),
            pltpu.VMEM((bpw, dim), jnp.float32),
            pltpu.SemaphoreType.DMA,
        ],
    )
    def k(uidx_h, usr_h, uv_h, idx_v, rows_f, sem):
        base = _wid() * bpw
        sl = pl.ds(base, bpw)
        pltpu.sync_copy(uidx_h.at[sl], idx_v)
        pltpu.async_copy(usr_h.at[idx_v], rows_f, sem).wait()
        pltpu.sync_copy(rows_f, uv_h.at[sl])

    return k(uidx, usr_emb)


def _sc_stage2(idx2d, adj_e, adj_r):
    nrows, W = idx2d.shape
    nn = adj_e.shape[1]
    rpw = nrows // NW
    nidx = nrows * W

    out_type = (
        jax.ShapeDtypeStruct((nidx, nn), jnp.int32),
        jax.ShapeDtypeStruct((nidx, nn), jnp.int32),
    )

    @functools.partial(
        pl.kernel, out_type=out_type, mesh=_sc_mesh(),
        compiler_params=_SC_PARAMS,
        scratch_types=[
            pltpu.VMEM((W,), jnp.int32),
            pltpu.VMEM((W, nn), jnp.int32),
            pltpu.SemaphoreType.DMA,
        ],
    )
    def k(idx_h, adj_e_h, adj_r_h, e2_h, r2_h, idx_v, rows_i, sem):
        w0 = _wid() * rpw

        def body(j, _):
            row = w0 + j
            osl = pl.ds(row * W, W)
            pltpu.sync_copy(idx_h.at[row], idx_v)
            pltpu.async_copy(adj_e_h.at[idx_v], rows_i, sem).wait()
            pltpu.sync_copy(rows_i, e2_h.at[osl])
            pltpu.async_copy(adj_r_h.at[idx_v], rows_i, sem).wait()
            pltpu.sync_copy(rows_i, r2_h.at[osl])
            return 0

        lax.fori_loop(0, rpw, body, 0)

    return k(idx2d, adj_e, adj_r)


def _sc_stage_late(head, iidx, e1_idx2d, ent_emb, itm_emb):
    Bn = head.shape[0]
    dim = ent_emb.shape[1]
    bpw = Bn // NW
    nrows, W = e1_idx2d.shape
    rpw = nrows // NW

    out_type = (
        jax.ShapeDtypeStruct((Bn, dim), jnp.float32),
        jax.ShapeDtypeStruct((nrows * W, dim), jnp.float32),
        jax.ShapeDtypeStruct((Bn, dim), jnp.float32),
    )

    @functools.partial(
        pl.kernel, out_type=out_type, mesh=_sc_mesh(),
        compiler_params=_SC_PARAMS,
        scratch_types=[
            pltpu.VMEM((bpw,), jnp.int32),
            pltpu.VMEM((W,), jnp.int32),
            pltpu.VMEM((W, dim), jnp.float32),
            pltpu.SemaphoreType.DMA,
        ],
    )
    def k(head_h, iidx_h, e1_h, ent_h, itm_h, ev0_h, ev1_h, iv_h,
          idx_v, idx2_v, rows_f, sem):
        wid = _wid()
        base = wid * bpw
        sl = pl.ds(base, bpw)
        pltpu.sync_copy(head_h.at[sl], idx_v)
        pltpu.async_copy(ent_h.at[idx_v], rows_f.at[pl.ds(0, bpw)], sem).wait()
        pltpu.sync_copy(rows_f.at[pl.ds(0, bpw)], ev0_h.at[sl])
        pltpu.sync_copy(iidx_h.at[sl], idx_v)
        pltpu.async_copy(itm_h.at[idx_v], rows_f.at[pl.ds(0, bpw)], sem).wait()
        pltpu.sync_copy(rows_f.at[pl.ds(0, bpw)], iv_h.at[sl])
        w0 = wid * rpw

        def body(j, _):
            row = w0 + j
            osl = pl.ds(row * W, W)
            pltpu.sync_copy(e1_h.at[row], idx2_v)
            pltpu.async_copy(ent_h.at[idx2_v], rows_f, sem).wait()
            pltpu.sync_copy(rows_f, ev1_h.at[osl])
            return 0

        lax.fori_loop(0, rpw, body, 0)

    return k(head, iidx, e1_idx2d, ent_emb, itm_emb)


def _sc_stage3(idx2d, w_flat, ent_emb):
    nrows, W = idx2d.shape
    dim = ent_emb.shape[1]
    nn = 16
    ng = w_flat.shape[0] // nn
    rpw = nrows // NW
    gpw = ng // NW
    gpc = W // nn
    OB = 16
    NBUF = 3

    out_type = jax.ShapeDtypeStruct((ng, dim), jnp.float32)

    @functools.partial(
        pl.kernel, out_type=out_type, mesh=_sc_mesh(),
        compiler_params=_SC_PARAMS_V,
        scratch_types=[
            pltpu.VMEM((rpw, W), jnp.int32),
            pltpu.VMEM((gpw * nn,), jnp.float32),
            pltpu.VMEM((NBUF, W, dim), jnp.float32),
            pltpu.VMEM((OB * gpc, dim), jnp.float32),
            pltpu.SemaphoreType.DMA,
            pltpu.SemaphoreType.DMA,
            pltpu.SemaphoreType.DMA,
            pltpu.SemaphoreType.DMA,
        ],
    )
    def k(idx_h, w_h, ent_h, out_h,
          idx_v, w_v, rows_v, out_v, sem0, sem1, sem2, semo):
        wid = _wid()
        w0 = wid * rpw
        pltpu.sync_copy(idx_h.at[pl.ds(w0, rpw)], idx_v)
        pltpu.sync_copy(w_h.at[pl.ds(wid * gpw * nn, gpw * nn)], w_v)
        sems = (sem0, sem1, sem2)

        def start(j, slot):
            return pltpu.async_copy(ent_h.at[idx_v.at[j]], rows_v.at[slot],
                                    sems[slot])

        for s in range(NBUF):
            start(s, s)

        def chunk_body(j, _):
            slot = lax.rem(j, NBUF)
            for s in range(NBUF):
                @pl.when(slot == s)
                def _():
                    pltpu.make_async_copy(ent_h.at[idx_v.at[j]],
                                          rows_v.at[s], sems[s]).wait()

            obase = lax.rem(j, OB) * gpc
            zi = jnp.zeros((16,), jnp.int32)
            for g in range(gpc):
                base = g * nn
                wbase = (j * gpc + g) * nn
                wn = [plsc.load_gather(w_v, [zi + (wbase + n)])
                      for n in range(nn)]
                for db in range(dim // 16):
                    sl = pl.ds(db * 16, 16)
                    t = [wn[n] * rows_v[slot, base + n, sl]
                         for n in range(nn)]
                    while len(t) > 1:
                        t = [t[k2] + t[k2 + 1] for k2 in range(0, len(t), 2)]
                    out_v[obase + g, sl] = t[0]

            for s in range(NBUF):
                @pl.when((slot == s) & (j + NBUF < rpw))
                def _():
                    start(j + NBUF, s)

            @pl.when(lax.rem(j, OB) == OB - 1)
            def _():
                pltpu.async_copy(
                    out_v,
                    out_h.at[pl.ds(wid * gpw + (j - (OB - 1)) * gpc,
                                   OB * gpc)],
                    semo).wait()

            return 0

        lax.fori_loop(0, rpw, chunk_body, 0)

    return k(idx2d, w_flat, ent_emb)


def _w2_kernel(r2p_r, p_r, rep2_r, w_r):
    YB = r2p_r.shape[0]
    nrel = p_r.shape[1]
    nn = 16
    r2p = r2p_r[...]
    Pexp = lax.dot_general(rep2_r[...], p_r[...], (((1,), (0,)), ((), ())),
                           preferred_element_type=jnp.float32)
    S = jnp.zeros(r2p.shape, jnp.float32)
    for r in range(nrel):
        S = jnp.where(r2p == r, Pexp[:, r:r + 1], S)
    m = jnp.max(S, axis=1, keepdims=True)
    e = jnp.exp(S - m)
    gl = lax.broadcasted_iota(jnp.int32, (128, 128), 0) // nn
    gc = lax.broadcasted_iota(jnp.int32, (128, 128), 1) // nn
    GS = (gl == gc).astype(jnp.float32)
    denom = lax.dot_general(e, GS, (((1,), (0,)), ((), ())),
                            preferred_element_type=jnp.float32)
    w_r[...] = e / (denom * float(nn))


def _p_kernel(uv_r, rel_r, p_r):
    dim = uv_r.shape[1]
    p_r[...] = lax.dot_general(uv_r[...], rel_r[...], (((1,), (1,)), ((), ())),
                               preferred_element_type=jnp.float32) * (1.0 / dim)


def _dense_kernel(uv_r, iv_r, ev0_r, ev1_r, agg1_r, r1_r, p_r, lab_r,
                  e3_r, w0bd_r, b0p_r, msum_r,
                  rel_r, umw_r, umb_r, wvv_r, wev_r, wve_r, wee_r, bv_r, be_r,
                  w0_r, b0_r, w1_r, b1_r,
                  sig_r, bce_r, l2_r):
    i = pl.program_id(0)
    BB = uv_r.shape[0]
    dim = uv_r.shape[1]
    nn = 16
    nrel = rel_r.shape[0]

    u = uv_r[...]
    rel = rel_r[...]
    P = p_r[...]

    r1i = r1_r[...]
    S1 = jnp.zeros((BB, nn), jnp.float32)
    for r in range(nrel):
        S1 = jnp.where(r1i == r, P[:, r:r + 1], S1)
    m = jnp.max(S1, axis=-1, keepdims=True)
    e = jnp.exp(S1 - m)
    w1 = e / jnp.sum(e, axis=-1, keepdims=True)

    W13 = lax.dot_general(w1, e3_r[...], (((1,), (0,)), ((), ())),
                          preferred_element_type=jnp.float32)

    W0 = w0_r[...]
    W0bd = w0bd_r[...]
    b0 = b0_r[...]
    b0p = b0p_r[...]

    ev1 = ev1_r[...]
    agg1p = agg1_r[...]
    h1p = jax.nn.relu(
        lax.dot_general(ev1 + agg1p, W0bd, (((1,), (0,)), ((), ())),
                        preferred_element_type=jnp.float32) + b0p)

    W13r = W13.reshape(BB * 8, 2 * dim)
    Msum = msum_r[...]

    def hop0_agg(xp):
        s = lax.dot_general(Msum, xp * W13r, (((1,), (0,)), ((), ())),
                            preferred_element_type=jnp.float32)
        return (s[:, :dim] + s[:, dim:]) * (1.0 / nn)

    agg0 = hop0_agg(ev1)
    h0 = jax.nn.relu(
        lax.dot_general(ev0_r[...] + agg0, W0, (((1,), (0,)), ((), ())),
                        preferred_element_type=jnp.float32) + b0)

    aggf = hop0_agg(h1p)
    head = jnp.tanh(
        lax.dot_general(h0 + aggf, w1_r[...], (((1,), (0,)), ((), ())),
                        preferred_element_type=jnp.float32) + b1_r[...])

    uo = jax.nn.relu(
        lax.dot_general(u, umw_r[...], (((1,), (0,)), ((), ())),
                        preferred_element_type=jnp.float32) + umb_r[...])

    iv = iv_r[...]
    a1 = jnp.sum(head * wvv_r[...], axis=1, keepdims=True)
    a2 = jnp.sum(iv * wev_r[...], axis=1, keepdims=True)
    v_out = iv * a1 + head * a2 + bv_r[...]

    s = jnp.sum(uo * v_out, axis=1)
    sig_r[...] = (1.0 / (1.0 + jnp.exp(-s))).reshape(sig_r.shape)

    lab = lab_r[...].reshape(BB)
    bce = jnp.maximum(s, 0.0) - s * lab + jnp.log1p(jnp.exp(-jnp.abs(s)))

    @pl.when(i == 0)
    def _():
        psq = (jnp.sum(rel * rel)
               + jnp.sum(umw_r[...] ** 2) + jnp.sum(umb_r[...] ** 2)
               + jnp.sum(wvv_r[...] ** 2) + jnp.sum(wev_r[...] ** 2)
               + jnp.sum(wve_r[...] ** 2) + jnp.sum(wee_r[...] ** 2)
               + jnp.sum(bv_r[...] ** 2) + jnp.sum(be_r[...] ** 2)
               + jnp.sum(W0 * W0) + jnp.sum(b0 * b0)
               + jnp.sum(w1_r[...] ** 2) + jnp.sum(b1_r[...] ** 2))
        bce_r[...] = jnp.zeros((1, 1), jnp.float32)
        l2_r[...] = psq.reshape(1, 1)

    bce_r[...] += jnp.sum(bce).reshape(1, 1)
    l2_r[...] += (jnp.sum(uo * uo) + jnp.sum(v_out * v_out)).reshape(1, 1)


def _table_l2_kernel(a_r, b_r, c_r, acc_r):
    i = pl.program_id(0)

    @pl.when(i == 0)
    def _():
        acc_r[...] = jnp.zeros((1, 1), jnp.float32)

    a = a_r[...]
    b = b_r[...]
    c = c_r[...]
    acc_r[...] += (jnp.sum(a * a) + jnp.sum(b * b) + jnp.sum(c * c)).reshape(1, 1)


def kernel(user_indices, item_indices, labels, head_indices, adj_entity,
           adj_relation, user_emb, item_emb, entity_emb, relation_emb,
           user_mlp_W, user_mlp_b, w_vv, w_ev, w_ve, w_ee, b_v, b_e,
           agg_W0, agg_b0, agg_W1, agg_b1):
    Bn = user_indices.shape[0]
    dim = user_emb.shape[1]
    nn = adj_entity.shape[1]
    nrel = relation_emb.shape[0]

    uv = _sc_stage_u(user_indices, user_emb)

    e1, r1 = _sc_stage1(head_indices, adj_entity, adj_relation)

    e1_idx = e1.reshape(Bn * nn // 128, 128)
    e2, r2 = _sc_stage2(e1_idx, adj_entity, adj_relation)

    PB = 512
    p_mat = pl.pallas_call(
        _p_kernel,
        grid=(Bn // PB,),
        in_specs=[
            pl.BlockSpec((PB, dim), lambda i: (i, 0)),
            pl.BlockSpec((nrel, dim), lambda i: (0, 0)),
        ],
        out_specs=pl.BlockSpec((PB, nrel), lambda i: (i, 0)),
        out_shape=jax.ShapeDtypeStruct((Bn, nrel), jnp.float32),
    )(uv, relation_emb)

    YB = 512
    r2_p = r2.reshape(Bn * nn * nn // 128, 128)
    yrows = lax.broadcasted_iota(jnp.int32, (YB, YB // 2), 0) // 2
    ycols = lax.broadcasted_iota(jnp.int32, (YB, YB // 2), 1)
    rep2 = (yrows == ycols).astype(jnp.float32)
    BBc = 64
    msb = lax.broadcasted_iota(jnp.int32, (BBc, BBc * 8), 0)
    mss = lax.broadcasted_iota(jnp.int32, (BBc, BBc * 8), 1) // 8
    msum_c = (msb == mss).astype(jnp.float32)
    w2_p = pl.pallas_call(
        _w2_kernel,
        grid=(Bn * nn * nn // 128 // YB,),
        in_specs=[
            pl.BlockSpec((YB, 128), lambda i: (i, 0)),
            pl.BlockSpec((YB // 2, nrel), lambda i: (i, 0)),
            pl.BlockSpec((YB, YB // 2), lambda i: (0, 0)),
        ],
        out_specs=pl.BlockSpec((YB, 128), lambda i: (i, 0)),
        out_shape=jax.ShapeDtypeStruct((Bn * nn * nn // 128, 128),
                                       jnp.float32),
    )(r2_p, p_mat, rep2)

    e2_idx = e2.reshape(Bn * nn * nn // 128, 128)
    agg1 = _sc_stage3(e2_idx, w2_p.reshape(Bn * nn * nn), entity_emb)

    ev0, ev1, iv = _sc_stage_late(head_indices, item_indices, e1_idx,
                                  entity_emb, item_emb)

    BB = 64
    grid = Bn // BB
    en = jnp.arange(nn)[:, None, None]
    esl = jnp.arange(8)[None, :, None] * 2 + jnp.arange(2 * dim)[None, None, :] // dim
    e3_c = (en == esl).astype(jnp.float32)
    z64 = jnp.zeros((dim, dim), jnp.float32)
    w0bd_c = jnp.block([[agg_W0, z64], [z64, agg_W0]])
    b0p_c = jnp.concatenate([agg_b0, agg_b0]).reshape(1, 2 * dim)
    ev1_p = ev1.reshape(Bn * nn * dim // 128, 128)
    agg1_p = agg1.reshape(Bn * nn * dim // 128, 128)
    lab_f = labels.astype(jnp.float32).reshape(grid, 1, BB)

    row = lambda x: x.reshape(1, dim)
    full = lambda shp: pl.BlockSpec(shp, lambda i: tuple(0 for _ in shp))

    sig, bce_sum, act_sq = pl.pallas_call(
        _dense_kernel,
        grid=(grid,),
        in_specs=[
            pl.BlockSpec((BB, dim), lambda i: (i, 0)),
            pl.BlockSpec((BB, dim), lambda i: (i, 0)),
            pl.BlockSpec((BB, dim), lambda i: (i, 0)),
            pl.BlockSpec((BB * nn * dim // 128, 128), lambda i: (i, 0)),
            pl.BlockSpec((BB * nn * dim // 128, 128), lambda i: (i, 0)),
            pl.BlockSpec((BB, nn), lambda i: (i, 0)),
            pl.BlockSpec((BB, nrel), lambda i: (i, 0)),
            pl.BlockSpec((1, 1, BB), lambda i: (i, 0, 0)),
            full((nn, 8, 2 * dim)),
            full((2 * dim, 2 * dim)),
            full((1, 2 * dim)),
            full((BB, BB * 8)),
            full((nrel, dim)),
            full((dim, dim)),
            full((1, dim)),
            full((1, dim)),
            full((1, dim)),
            full((1, dim)),
            full((1, dim)),
            full((1, dim)),
            full((1, dim)),
            full((dim, dim)),
            full((1, dim)),
            full((dim, dim)),
            full((1, dim)),
        ],
        out_specs=[
            pl.BlockSpec((1, 1, BB), lambda i: (i, 0, 0)),
            pl.BlockSpec((1, 1), lambda i: (0, 0)),
            pl.BlockSpec((1, 1), lambda i: (0, 0)),
        ],
        out_shape=[
            jax.ShapeDtypeStruct((grid, 1, BB), jnp.float32),
            jax.ShapeDtypeStruct((1, 1), jnp.float32),
            jax.ShapeDtypeStruct((1, 1), jnp.float32),
        ],
    )(uv, iv, ev0, ev1_p, agg1_p, r1, p_mat, lab_f,
      e3_c, w0bd_c, b0p_c, msum_c,
      relation_emb, user_mlp_W, row(user_mlp_b),
      row(w_vv.reshape(dim)), row(w_ev.reshape(dim)),
      row(w_ve.reshape(dim)), row(w_ee.reshape(dim)),
      row(b_v), row(b_e),
      agg_W0, row(agg_b0), agg_W1, row(agg_b1))

    RB = 2000
    tgrid = user_emb.shape[0] // RB
    tab_sq = pl.pallas_call(
        _table_l2_kernel,
        grid=(tgrid,),
        in_specs=[
            pl.BlockSpec((RB, dim), lambda i: (i, 0)),
            pl.BlockSpec((RB, dim), lambda i: (i, 0)),
            pl.BlockSpec((RB, dim), lambda i: (i, 0)),
        ],
        out_specs=pl.BlockSpec((1, 1), lambda i: (0, 0)),
        out_shape=jax.ShapeDtypeStruct((1, 1), jnp.float32),
    )(user_emb, item_emb, entity_emb)

    scores_normalized = sig.reshape(Bn)
    total_sq = act_sq[0, 0] + tab_sq[0, 0]
    loss = bce_sum[0, 0] / Bn + (0.5 * L2W) * total_sq
    return (scores_normalized, loss)


# speedup vs baseline: 1.0110x; 1.0110x over previous
import functools

import jax
import jax.numpy as jnp
from jax import lax
from jax.experimental import pallas as pl
from jax.experimental.pallas import tpu as pltpu
from jax.experimental.pallas import tpu_sc as plsc

NC, NS = 2, 16
NW = NC * NS
L2W = 1e-06


def _sc_mesh():
    return plsc.VectorSubcoreMesh(core_axis_name="c", subcore_axis_name="s",
                                  num_cores=NC, num_subcores=NS)


_SC_PARAMS = pltpu.CompilerParams(use_tc_tiling_on_sc=False)
_SC_PARAMS_V = pltpu.CompilerParams(use_tc_tiling_on_sc=False,
                                    needs_layout_passes=False)


def _wid():
    return lax.axis_index("s") * NC + lax.axis_index("c")


def _sc_stage1(head, adj_e, adj_r):
    Bn = head.shape[0]
    nn = adj_e.shape[1]
    bpw = Bn // NW

    out_type = (
        jax.ShapeDtypeStruct((Bn, nn), jnp.int32),
        jax.ShapeDtypeStruct((Bn, nn), jnp.int32),
    )

    @functools.partial(
        pl.kernel, out_type=out_type, mesh=_sc_mesh(),
        compiler_params=_SC_PARAMS,
        scratch_types=[
            pltpu.VMEM((bpw,), jnp.int32),
            pltpu.VMEM((bpw, nn), jnp.int32),
            pltpu.SemaphoreType.DMA,
        ],
    )
    def k(head_h, adj_e_h, adj_r_h, e1_h, r1_h, idx_v, rows_i, sem):
        base = _wid() * bpw
        sl = pl.ds(base, bpw)
        pltpu.sync_copy(head_h.at[sl], idx_v)
        pltpu.async_copy(adj_e_h.at[idx_v], rows_i, sem).wait()
        pltpu.sync_copy(rows_i, e1_h.at[sl])
        pltpu.async_copy(adj_r_h.at[idx_v], rows_i, sem).wait()
        pltpu.sync_copy(rows_i, r1_h.at[sl])

    return k(head, adj_e, adj_r)


def _sc_stage_u(uidx, usr_emb):
    Bn = uidx.shape[0]
    dim = usr_emb.shape[1]
    bpw = Bn // NW

    out_type = jax.ShapeDtypeStruct((Bn, dim), jnp.float32)

    @functools.partial(
        pl.kernel, out_type=out_type, mesh=_sc_mesh(),
        compiler_params=_SC_PARAMS,
        scratch_types=[
            pltpu.VMEM((bpw,), jnp.int32),
            pltpu.VMEM((bpw, dim), jnp.float32),
            pltpu.SemaphoreType.DMA,
        ],
    )
    def k(uidx_h, usr_h, uv_h, idx_v, rows_f, sem):
        base = _wid() * bpw
        sl = pl.ds(base, bpw)
        pltpu.sync_copy(uidx_h.at[sl], idx_v)
        pltpu.async_copy(usr_h.at[idx_v], rows_f, sem).wait()
        pltpu.sync_copy(rows_f, uv_h.at[sl])

    return k(uidx, usr_emb)


def _sc_stage2(idx2d, adj_e, adj_r):
    nrows, W = idx2d.shape
    nn = adj_e.shape[1]
    rpw = nrows // NW
    nidx = nrows * W

    out_type = (
        jax.ShapeDtypeStruct((nidx, nn), jnp.int32),
        jax.ShapeDtypeStruct((nidx, nn), jnp.int32),
    )

    @functools.partial(
        pl.kernel, out_type=out_type, mesh=_sc_mesh(),
        compiler_params=_SC_PARAMS,
        scratch_types=[
            pltpu.VMEM((W,), jnp.int32),
            pltpu.VMEM((W, nn), jnp.int32),
            pltpu.SemaphoreType.DMA,
        ],
    )
    def k(idx_h, adj_e_h, adj_r_h, e2_h, r2_h, idx_v, rows_i, sem):
        w0 = _wid() * rpw

        def body(j, _):
            row = w0 + j
            osl = pl.ds(row * W, W)
            pltpu.sync_copy(idx_h.at[row], idx_v)
            pltpu.async_copy(adj_e_h.at[idx_v], rows_i, sem).wait()
            pltpu.sync_copy(rows_i, e2_h.at[osl])
            pltpu.async_copy(adj_r_h.at[idx_v], rows_i, sem).wait()
            pltpu.sync_copy(rows_i, r2_h.at[osl])
            return 0

        lax.fori_loop(0, rpw, body, 0)

    return k(idx2d, adj_e, adj_r)


def _sc_stage_late(head, iidx, e1_idx2d, ent_emb, itm_emb):
    Bn = head.shape[0]
    dim = ent_emb.shape[1]
    bpw = Bn // NW
    nrows, W = e1_idx2d.shape
    rpw = nrows // NW

    out_type = (
        jax.ShapeDtypeStruct((Bn, dim), jnp.float32),
        jax.ShapeDtypeStruct((nrows * W, dim), jnp.float32),
        jax.ShapeDtypeStruct((Bn, dim), jnp.float32),
    )

    @functools.partial(
        pl.kernel, out_type=out_type, mesh=_sc_mesh(),
        compiler_params=_SC_PARAMS,
        scratch_types=[
            pltpu.VMEM((bpw,), jnp.int32),
            pltpu.VMEM((W,), jnp.int32),
            pltpu.VMEM((W, dim), jnp.float32),
            pltpu.SemaphoreType.DMA,
        ],
    )
    def k(head_h, iidx_h, e1_h, ent_h, itm_h, ev0_h, ev1_h, iv_h,
          idx_v, idx2_v, rows_f, sem):
        wid = _wid()
        base = wid * bpw
        sl = pl.ds(base, bpw)
        pltpu.sync_copy(head_h.at[sl], idx_v)
        pltpu.async_copy(ent_h.at[idx_v], rows_f.at[pl.ds(0, bpw)], sem).wait()
        pltpu.sync_copy(rows_f.at[pl.ds(0, bpw)], ev0_h.at[sl])
        pltpu.sync_copy(iidx_h.at[sl], idx_v)
        pltpu.async_copy(itm_h.at[idx_v], rows_f.at[pl.ds(0, bpw)], sem).wait()
        pltpu.sync_copy(rows_f.at[pl.ds(0, bpw)], iv_h.at[sl])
        w0 = wid * rpw

        def body(j, _):
            row = w0 + j
            osl = pl.ds(row * W, W)
            pltpu.sync_copy(e1_h.at[row], idx2_v)
            pltpu.async_copy(ent_h.at[idx2_v], rows_f, sem).wait()
            pltpu.sync_copy(rows_f, ev1_h.at[osl])
            return 0

        lax.fori_loop(0, rpw, body, 0)

    return k(head, iidx, e1_idx2d, ent_emb, itm_emb)


def _sc_stage3(idx2d, w_flat, ent_emb):
    nrows, W = idx2d.shape
    dim = ent_emb.shape[1]
    nn = 16
    ng = w_flat.shape[0] // nn
    rpw = nrows // NW
    gpw = ng // NW
    gpc = W // nn
    OB = 16
    NBUF = 3

    out_type = jax.ShapeDtypeStruct((ng, dim), jnp.float32)

    @functools.partial(
        pl.kernel, out_type=out_type, mesh=_sc_mesh(),
        compiler_params=_SC_PARAMS_V,
        scratch_types=[
            pltpu.VMEM((rpw, W), jnp.int32),
            pltpu.VMEM((gpw * nn,), jnp.float32),
            pltpu.VMEM((NBUF, W, dim), jnp.float32),
            pltpu.VMEM((OB * gpc, dim), jnp.float32),
            pltpu.SemaphoreType.DMA,
            pltpu.SemaphoreType.DMA,
            pltpu.SemaphoreType.DMA,
            pltpu.SemaphoreType.DMA,
        ],
    )
    def k(idx_h, w_h, ent_h, out_h,
          idx_v, w_v, rows_v, out_v, sem0, sem1, sem2, semo):
        wid = _wid()
        w0 = wid * rpw
        pltpu.sync_copy(idx_h.at[pl.ds(w0, rpw)], idx_v)
        pltpu.sync_copy(w_h.at[pl.ds(wid * gpw * nn, gpw * nn)], w_v)
        sems = (sem0, sem1, sem2)

        def start(j, slot):
            return pltpu.async_copy(ent_h.at[idx_v.at[j]], rows_v.at[slot],
                                    sems[slot])

        for s in range(NBUF):
            start(s, s)

        def chunk_body(j, _):
            slot = lax.rem(j, NBUF)
            for s in range(NBUF):
                @pl.when(slot == s)
                def _():
                    pltpu.make_async_copy(ent_h.at[idx_v.at[j]],
                                          rows_v.at[s], sems[s]).wait()

            obase = lax.rem(j, OB) * gpc
            zi = jnp.zeros((16,), jnp.int32)
            for g in range(gpc):
                base = g * nn
                wbase = (j * gpc + g) * nn
                wn = [plsc.load_gather(w_v, [zi + (wbase + n)])
                      for n in range(nn)]
                for db in range(dim // 16):
                    sl = pl.ds(db * 16, 16)
                    t = [wn[n] * rows_v[slot, base + n, sl]
                         for n in range(nn)]
                    while len(t) > 1:
                        t = [t[k2] + t[k2 + 1] for k2 in range(0, len(t), 2)]
                    out_v[obase + g, sl] = t[0]

            for s in range(NBUF):
                @pl.when((slot == s) & (j + NBUF < rpw))
                def _():
                    start(j + NBUF, s)

            @pl.when(lax.rem(j, OB) == OB - 1)
            def _():
                pltpu.async_copy(
                    out_v,
                    out_h.at[pl.ds(wid * gpw + (j - (OB - 1)) * gpc,
                                   OB * gpc)],
                    semo).wait()

            return 0

        lax.fori_loop(0, rpw, chunk_body, 0)

    return k(idx2d, w_flat, ent_emb)


def _w2_kernel(r2p_r, p_r, rep2_r, w_r):
    YB = r2p_r.shape[0]
    nrel = p_r.shape[1]
    nn = 16
    r2p = r2p_r[...]
    Pexp = lax.dot_general(rep2_r[...], p_r[...], (((1,), (0,)), ((), ())),
                           preferred_element_type=jnp.float32)
    S = jnp.zeros(r2p.shape, jnp.float32)
    for r in range(nrel):
        S = jnp.where(r2p == r, Pexp[:, r:r + 1], S)
    m = jnp.max(S, axis=1, keepdims=True)
    e = jnp.exp(S - m)
    gl = lax.broadcasted_iota(jnp.int32, (128, 128), 0) // nn
    gc = lax.broadcasted_iota(jnp.int32, (128, 128), 1) // nn
    GS = (gl == gc).astype(jnp.float32)
    denom = lax.dot_general(e, GS, (((1,), (0,)), ((), ())),
                            preferred_element_type=jnp.float32)
    w_r[...] = e / (denom * float(nn))


def _p_kernel(uv_r, rel_r, p_r):
    dim = uv_r.shape[1]
    p_r[...] = lax.dot_general(uv_r[...], rel_r[...], (((1,), (1,)), ((), ())),
                               preferred_element_type=jnp.float32) * (1.0 / dim)


def _dense_kernel(uv_r, iv_r, ev0_r, ev1_r, agg1_r, r1_r, p_r, lab_r,
                  e3_r, w0bd_r, b0p_r,
                  rel_r, umw_r, umb_r, wvv_r, wev_r, wve_r, wee_r, bv_r, be_r,
                  w0_r, b0_r, w1_r, b1_r,
                  sig_r, bce_r, l2_r):
    i = pl.program_id(0)
    BB = uv_r.shape[0]
    dim = uv_r.shape[1]
    nn = 16
    nrel = rel_r.shape[0]

    u = uv_r[...]
    rel = rel_r[...]
    P = p_r[...]

    r1i = r1_r[...]
    S1 = jnp.zeros((BB, nn), jnp.float32)
    for r in range(nrel):
        S1 = jnp.where(r1i == r, P[:, r:r + 1], S1)
    m = jnp.max(S1, axis=-1, keepdims=True)
    e = jnp.exp(S1 - m)
    w1 = e / jnp.sum(e, axis=-1, keepdims=True)

    W13 = lax.dot_general(w1, e3_r[...], (((1,), (0,)), ((), ())),
                          preferred_element_type=jnp.float32)

    W0 = w0_r[...]
    W0bd = w0bd_r[...]
    b0 = b0_r[...]
    b0p = b0p_r[...]

    ev1 = ev1_r[...]
    agg1p = agg1_r[...]
    h1p = jax.nn.relu(
        lax.dot_general(ev1 + agg1p, W0bd, (((1,), (0,)), ((), ())),
                        preferred_element_type=jnp.float32) + b0p)

    def hop0_agg(xp):
        s = jnp.sum(xp.reshape(BB, 8, 2 * dim) * W13, axis=1)
        return (s[:, :dim] + s[:, dim:]) * (1.0 / nn)

    agg0 = hop0_agg(ev1)
    h0 = jax.nn.relu(
        lax.dot_general(ev0_r[...] + agg0, W0, (((1,), (0,)), ((), ())),
                        preferred_element_type=jnp.float32) + b0)

    aggf = hop0_agg(h1p)
    head = jnp.tanh(
        lax.dot_general(h0 + aggf, w1_r[...], (((1,), (0,)), ((), ())),
                        preferred_element_type=jnp.float32) + b1_r[...])

    uo = jax.nn.relu(
        lax.dot_general(u, umw_r[...], (((1,), (0,)), ((), ())),
                        preferred_element_type=jnp.float32) + umb_r[...])

    iv = iv_r[...]
    a1 = jnp.sum(head * wvv_r[...], axis=1, keepdims=True)
    a2 = jnp.sum(iv * wev_r[...], axis=1, keepdims=True)
    v_out = iv * a1 + head * a2 + bv_r[...]

    s = jnp.sum(uo * v_out, axis=1)
    sig_r[...] = (1.0 / (1.0 + jnp.exp(-s))).reshape(sig_r.shape)

    lab = lab_r[...].reshape(BB)
    bce = jnp.maximum(s, 0.0) - s * lab + jnp.log1p(jnp.exp(-jnp.abs(s)))

    @pl.when(i == 0)
    def _():
        psq = (jnp.sum(rel * rel)
               + jnp.sum(umw_r[...] ** 2) + jnp.sum(umb_r[...] ** 2)
               + jnp.sum(wvv_r[...] ** 2) + jnp.sum(wev_r[...] ** 2)
               + jnp.sum(wve_r[...] ** 2) + jnp.sum(wee_r[...] ** 2)
               + jnp.sum(bv_r[...] ** 2) + jnp.sum(be_r[...] ** 2)
               + jnp.sum(W0 * W0) + jnp.sum(b0 * b0)
               + jnp.sum(w1_r[...] ** 2) + jnp.sum(b1_r[...] ** 2))
        bce_r[...] = jnp.zeros((1, 1), jnp.float32)
        l2_r[...] = psq.reshape(1, 1)

    bce_r[...] += jnp.sum(bce).reshape(1, 1)
    l2_r[...] += (jnp.sum(uo * uo) + jnp.sum(v_out * v_out)).reshape(1, 1)


def _table_l2_kernel(a_r, b_r, c_r, acc_r):
    i = pl.program_id(0)

    @pl.when(i == 0)
    def _():
        acc_r[...] = jnp.zeros((1, 1), jnp.float32)

    a = a_r[...]
    b = b_r[...]
    c = c_r[...]
    acc_r[...] += (jnp.sum(a * a) + jnp.sum(b * b) + jnp.sum(c * c)).reshape(1, 1)


def kernel(user_indices, item_indices, labels, head_indices, adj_entity,
           adj_relation, user_emb, item_emb, entity_emb, relation_emb,
           user_mlp_W, user_mlp_b, w_vv, w_ev, w_ve, w_ee, b_v, b_e,
           agg_W0, agg_b0, agg_W1, agg_b1):
    Bn = user_indices.shape[0]
    dim = user_emb.shape[1]
    nn = adj_entity.shape[1]
    nrel = relation_emb.shape[0]

    uv = _sc_stage_u(user_indices, user_emb)

    e1, r1 = _sc_stage1(head_indices, adj_entity, adj_relation)

    e1_idx = e1.reshape(Bn * nn // 128, 128)
    e2, r2 = _sc_stage2(e1_idx, adj_entity, adj_relation)

    PB = 512
    p_mat = pl.pallas_call(
        _p_kernel,
        grid=(Bn // PB,),
        in_specs=[
            pl.BlockSpec((PB, dim), lambda i: (i, 0)),
            pl.BlockSpec((nrel, dim), lambda i: (0, 0)),
        ],
        out_specs=pl.BlockSpec((PB, nrel), lambda i: (i, 0)),
        out_shape=jax.ShapeDtypeStruct((Bn, nrel), jnp.float32),
    )(uv, relation_emb)

    YB = 512
    r2_p = r2.reshape(Bn * nn * nn // 128, 128)
    yrows = lax.broadcasted_iota(jnp.int32, (YB, YB // 2), 0) // 2
    ycols = lax.broadcasted_iota(jnp.int32, (YB, YB // 2), 1)
    rep2 = (yrows == ycols).astype(jnp.float32)
    w2_p = pl.pallas_call(
        _w2_kernel,
        grid=(Bn * nn * nn // 128 // YB,),
        in_specs=[
            pl.BlockSpec((YB, 128), lambda i: (i, 0)),
            pl.BlockSpec((YB // 2, nrel), lambda i: (i, 0)),
            pl.BlockSpec((YB, YB // 2), lambda i: (0, 0)),
        ],
        out_specs=pl.BlockSpec((YB, 128), lambda i: (i, 0)),
        out_shape=jax.ShapeDtypeStruct((Bn * nn * nn // 128, 128),
                                       jnp.float32),
    )(r2_p, p_mat, rep2)

    e2_idx = e2.reshape(Bn * nn * nn // 128, 128)
    agg1 = _sc_stage3(e2_idx, w2_p.reshape(Bn * nn * nn), entity_emb)

    ev0, ev1, iv = _sc_stage_late(head_indices, item_indices, e1_idx,
                                  entity_emb, item_emb)

    BB = 64
    grid = Bn // BB
    en = jnp.arange(nn)[:, None, None]
    esl = jnp.arange(8)[None, :, None] * 2 + jnp.arange(2 * dim)[None, None, :] // dim
    e3_c = (en == esl).astype(jnp.float32)
    z64 = jnp.zeros((dim, dim), jnp.float32)
    w0bd_c = jnp.block([[agg_W0, z64], [z64, agg_W0]])
    b0p_c = jnp.concatenate([agg_b0, agg_b0]).reshape(1, 2 * dim)
    ev1_p = ev1.reshape(Bn * nn * dim // 128, 128)
    agg1_p = agg1.reshape(Bn * nn * dim // 128, 128)
    lab_f = labels.astype(jnp.float32).reshape(grid, 1, BB)

    row = lambda x: x.reshape(1, dim)
    full = lambda shp: pl.BlockSpec(shp, lambda i: tuple(0 for _ in shp))

    sig, bce_sum, act_sq = pl.pallas_call(
        _dense_kernel,
        grid=(grid,),
        in_specs=[
            pl.BlockSpec((BB, dim), lambda i: (i, 0)),
            pl.BlockSpec((BB, dim), lambda i: (i, 0)),
            pl.BlockSpec((BB, dim), lambda i: (i, 0)),
            pl.BlockSpec((BB * nn * dim // 128, 128), lambda i: (i, 0)),
            pl.BlockSpec((BB * nn * dim // 128, 128), lambda i: (i, 0)),
            pl.BlockSpec((BB, nn), lambda i: (i, 0)),
            pl.BlockSpec((BB, nrel), lambda i: (i, 0)),
            pl.BlockSpec((1, 1, BB), lambda i: (i, 0, 0)),
            full((nn, 8, 2 * dim)),
            full((2 * dim, 2 * dim)),
            full((1, 2 * dim)),
            full((nrel, dim)),
            full((dim, dim)),
            full((1, dim)),
            full((1, dim)),
            full((1, dim)),
            full((1, dim)),
            full((1, dim)),
            full((1, dim)),
            full((1, dim)),
            full((dim, dim)),
            full((1, dim)),
            full((dim, dim)),
            full((1, dim)),
        ],
        out_specs=[
            pl.BlockSpec((1, 1, BB), lambda i: (i, 0, 0)),
            pl.BlockSpec((1, 1), lambda i: (0, 0)),
            pl.BlockSpec((1, 1), lambda i: (0, 0)),
        ],
        out_shape=[
            jax.ShapeDtypeStruct((grid, 1, BB), jnp.float32),
            jax.ShapeDtypeStruct((1, 1), jnp.float32),
            jax.ShapeDtypeStruct((1, 1), jnp.float32),
        ],
    )(uv, iv, ev0, ev1_p, agg1_p, r1, p_mat, lab_f,
      e3_c, w0bd_c, b0p_c,
      relation_emb, user_mlp_W, row(user_mlp_b),
      row(w_vv.reshape(dim)), row(w_ev.reshape(dim)),
      row(w_ve.reshape(dim)), row(w_ee.reshape(dim)),
      row(b_v), row(b_e),
      agg_W0, row(agg_b0), agg_W1, row(agg_b1))

    RB = 2000
    tgrid = user_emb.shape[0] // RB
    tab_sq = pl.pallas_call(
        _table_l2_kernel,
        grid=(tgrid,),
        in_specs=[
            pl.BlockSpec((RB, dim), lambda i: (i, 0)),
            pl.BlockSpec((RB, dim), lambda i: (i, 0)),
            pl.BlockSpec((RB, dim), lambda i: (i, 0)),
        ],
        out_specs=pl.BlockSpec((1, 1), lambda i: (0, 0)),
        out_shape=jax.ShapeDtypeStruct((1, 1), jnp.float32),
    )(user_emb, item_emb, entity_emb)

    scores_normalized = sig.reshape(Bn)
    total_sq = act_sq[0, 0] + tab_sq[0, 0]
    loss = bce_sum[0, 0] / Bn + (0.5 * L2W) * total_sq
    return (scores_normalized, loss)
